# Initial kernel scaffold; baseline (speedup 1.0000x reference)
import jax
import jax.numpy as jnp
from jax.experimental import pallas as pl


def kernel(x, adj, cached_adj, Ws, bs):
    raise NotImplementedError("write your pallas kernel here")


# speedup vs baseline: 43.2160x; 43.2160x over previous
import functools

import jax
import jax.numpy as jnp
from jax import lax
from jax.experimental import pallas as pl
from jax.experimental.pallas import tpu as pltpu, tpu_sc as plsc

N = 4096
E = 65536
M = 2
D = 128
F = M * D
BLK = 256
NBLK = N // BLK
NW = 32
CPT = F // NW
EPW = E // NW
CH = 4096

_mesh = plsc.VectorSubcoreMesh(core_axis_name="c", subcore_axis_name="s")
_sc_params = pltpu.CompilerParams(needs_layout_passes=False)


@functools.partial(
    pl.kernel,
    out_type=jax.ShapeDtypeStruct((NW, N), jnp.float32),
    mesh=_mesh,
    compiler_params=_sc_params,
    scratch_types=[
        pltpu.VMEM((N,), jnp.float32),
        pltpu.VMEM((EPW,), jnp.int32),
    ],
)
def _deg_k(dst_hbm, zeros_hbm, out_hbm, hist_v, dbuf):
    c = lax.axis_index("c")
    s = lax.axis_index("s")
    wid = c * 16 + s
    pltpu.sync_copy(zeros_hbm, hist_v)
    pltpu.sync_copy(dst_hbm.at[pl.ds(wid * EPW, EPW)], dbuf)
    ones = jnp.ones((16,), jnp.float32)
    io = lax.broadcasted_iota(jnp.int32, (16,), 0)
    masks = [io == l for l in range(16)]

    def body(g, carry):
        d16 = dbuf[pl.ds(g * 16, 16)]
        for l in range(16):
            plsc.addupdate_scatter(hist_v, [d16], ones, mask=masks[l])
        return carry

    lax.fori_loop(0, EPW // 16, body, 0)
    pltpu.sync_copy(hist_v, out_hbm.at[wid])


@functools.partial(
    pl.kernel,
    out_type=jax.ShapeDtypeStruct((F, N), jnp.float32),
    mesh=_mesh,
    compiler_params=_sc_params,
    scratch_types=[
        pltpu.VMEM((CPT, N), jnp.float32),
        pltpu.VMEM((CPT, N), jnp.float32),
        pltpu.VMEM((CH,), jnp.int32),
        pltpu.VMEM((CH,), jnp.int32),
    ],
)
def _agg_k(src_hbm, dst_hbm, yst_hbm, zeros_hbm, out_hbm, ys_v, acc_v, sbuf, dbuf):
    c = lax.axis_index("c")
    s = lax.axis_index("s")
    wid = c * 16 + s
    pltpu.sync_copy(yst_hbm.at[pl.ds(wid * CPT, CPT)], ys_v)
    pltpu.sync_copy(zeros_hbm, acc_v)
    io = lax.broadcasted_iota(jnp.int32, (16,), 0)
    mlow = io < 8
    mhigh = io >= 8
    io7 = io & 7

    def chunk(ch, carry):
        pltpu.sync_copy(src_hbm.at[pl.ds(ch * CH, CH)], sbuf)
        pltpu.sync_copy(dst_hbm.at[pl.ds(ch * CH, CH)], dbuf)

        def body(g, carry2):
            s16 = sbuf[pl.ds(g * 16, 16)]
            d16 = dbuf[pl.ds(g * 16, 16)]
            for cc in range(CPT):
                colv = (cc + io7) & 7
                v = plsc.load_gather(ys_v, [colv, s16])
                plsc.addupdate_scatter(acc_v, [colv, d16], v, mask=mlow)
                plsc.addupdate_scatter(acc_v, [colv, d16], v, mask=mhigh)
            return carry2

        lax.fori_loop(0, CH // 16, body, 0)
        return carry

    lax.fori_loop(0, E // CH, chunk, 0)
    pltpu.sync_copy(acc_v, out_hbm.at[pl.ds(wid * CPT, CPT)])


def _dinv_diag(hist_blk):
    deg = jnp.sum(hist_blk, axis=0, keepdims=True) + 1.0
    dinv = lax.rsqrt(deg)
    r = lax.broadcasted_iota(jnp.int32, (BLK, BLK), 0)
    col = lax.broadcasted_iota(jnp.int32, (BLK, BLK), 1)
    return jnp.where(r == col, jnp.broadcast_to(dinv, (BLK, BLK)), 0.0)


_NT = (((1,), (1,)), ((), ()))


def _mm_t_body(xft_ref, a_ref, o_ref):
    o_ref[...] = lax.dot_general(xft_ref[...], a_ref[...], _NT,
                                 preferred_element_type=jnp.float32)


def _mm_t(xft, a):
    return pl.pallas_call(
        _mm_t_body,
        grid=(NBLK,),
        in_specs=[
            pl.BlockSpec((F, N), lambda i: (0, 0)),
            pl.BlockSpec((BLK, N), lambda i: (i, 0)),
        ],
        out_specs=pl.BlockSpec((F, BLK), lambda i: (0, i)),
        out_shape=jax.ShapeDtypeStruct((F, N), jnp.float32),
    )(xft, a)


def _stage_b_body(a1_ref, x1t_ref, xft_ref, hist_ref, w0t_ref, w1t_ref, o_ref):
    i = pl.program_id(0)
    x1t_full = x1t_ref[...]
    x1t_blk = x1t_ref[:, pl.ds(i * BLK, BLK)]
    x2t_blk = lax.dot_general(x1t_full, a1_ref[...], _NT,
                              preferred_element_type=jnp.float32)
    yt = (jnp.dot(w0t_ref[...], xft_ref[...] + x1t_blk,
                  preferred_element_type=jnp.float32)
          + jnp.dot(w1t_ref[...], x2t_blk, preferred_element_type=jnp.float32))
    o_ref[...] = jnp.dot(yt, _dinv_diag(hist_ref[...]),
                         preferred_element_type=jnp.float32)


def _stage_b(a1, x1t, xft, hist, w0t, w1t):
    return pl.pallas_call(
        _stage_b_body,
        grid=(NBLK,),
        in_specs=[
            pl.BlockSpec((BLK, N), lambda i: (i, 0)),
            pl.BlockSpec((F, N), lambda i: (0, 0)),
            pl.BlockSpec((F, BLK), lambda i: (0, i)),
            pl.BlockSpec((NW, BLK), lambda i: (0, i)),
            pl.BlockSpec((F, F), lambda i: (0, 0)),
            pl.BlockSpec((F, F), lambda i: (0, 0)),
        ],
        out_specs=pl.BlockSpec((F, BLK), lambda i: (0, i)),
        out_shape=jax.ShapeDtypeStruct((F, N), jnp.float32),
    )(a1, x1t, xft, hist, w0t, w1t)


def _combine_body(st_ref, yst_ref, hist_ref, b_ref, o_ref):
    sum_t = st_ref[...] + yst_ref[...]
    scaled = jnp.dot(sum_t, _dinv_diag(hist_ref[...]),
                     preferred_element_type=jnp.float32)
    o_ref[...] = jnp.transpose(scaled, (1, 0)) + b_ref[...]


def _combine(st, yst, hist, bias_row):
    return pl.pallas_call(
        _combine_body,
        grid=(NBLK,),
        in_specs=[
            pl.BlockSpec((F, BLK), lambda i: (0, i)),
            pl.BlockSpec((F, BLK), lambda i: (0, i)),
            pl.BlockSpec((NW, BLK), lambda i: (0, i)),
            pl.BlockSpec((1, F), lambda i: (0, 0)),
        ],
        out_specs=pl.BlockSpec((BLK, F), lambda i: (i, 0)),
        out_shape=jax.ShapeDtypeStruct((N, F), jnp.float32),
    )(st, yst, hist, bias_row)


def kernel(x, adj, cached_adj, Ws, bs):
    xft = x.reshape(N, F).T
    src = adj[0]
    dst = adj[1]
    eye_m = jnp.eye(M, dtype=jnp.float32)
    w0t = jnp.kron(eye_m, Ws[0].T)
    w1t = jnp.kron(eye_m, Ws[1].T)
    bias_row = jnp.tile(2.0 * bs[0] + bs[1], M).reshape(1, F)
    zeros_n = jnp.zeros((N,), jnp.float32)
    zeros_cpn = jnp.zeros((CPT, N), jnp.float32)

    hist = _deg_k(dst, zeros_n)
    x1t = _mm_t(xft, cached_adj[0])
    yst = _stage_b(cached_adj[1], x1t, xft, hist, w0t, w1t)
    st = _agg_k(src, dst, yst, zeros_cpn)
    out = _combine(st, yst, hist, bias_row)
    return out.reshape(N, M, D)


# speedup vs baseline: 45.6585x; 1.0565x over previous
import functools

import jax
import jax.numpy as jnp
from jax import lax
from jax.experimental import pallas as pl
from jax.experimental.pallas import tpu as pltpu, tpu_sc as plsc

N = 4096
E = 65536
M = 2
D = 128
F = M * D
BLK = 256
NBLK = N // BLK
NW = 32
CPT = F // NW
EPW = E // NW
CH = 8192
GU = 2

_mesh = plsc.VectorSubcoreMesh(core_axis_name="c", subcore_axis_name="s")
_sc_params = pltpu.CompilerParams(needs_layout_passes=False)


@functools.partial(
    pl.kernel,
    out_type=jax.ShapeDtypeStruct((NW, N), jnp.float32),
    mesh=_mesh,
    compiler_params=_sc_params,
    scratch_types=[
        pltpu.VMEM((N,), jnp.float32),
        pltpu.VMEM((EPW,), jnp.int32),
    ],
)
def _deg_k(dst_hbm, zeros_hbm, out_hbm, hist_v, dbuf):
    c = lax.axis_index("c")
    s = lax.axis_index("s")
    wid = c * 16 + s
    pltpu.sync_copy(zeros_hbm, hist_v)
    pltpu.sync_copy(dst_hbm.at[pl.ds(wid * EPW, EPW)], dbuf)
    ones = jnp.ones((16,), jnp.float32)
    io = lax.broadcasted_iota(jnp.int32, (16,), 0)
    masks = [io == l for l in range(16)]

    def body(g, carry):
        d16 = dbuf[pl.ds(g * 16, 16)]
        for l in range(16):
            plsc.addupdate_scatter(hist_v, [d16], ones, mask=masks[l])
        return carry

    lax.fori_loop(0, EPW // 16, body, 0)
    pltpu.sync_copy(hist_v, out_hbm.at[wid])


@functools.partial(
    pl.kernel,
    out_type=jax.ShapeDtypeStruct((F, N), jnp.float32),
    mesh=_mesh,
    compiler_params=_sc_params,
    scratch_types=[
        pltpu.VMEM((CPT, N), jnp.float32),
        pltpu.VMEM((2 * CPT, N), jnp.float32),
        pltpu.VMEM((CH,), jnp.int32),
        pltpu.VMEM((CH,), jnp.int32),
    ],
)
def _agg_k(src_hbm, dst_hbm, yst_hbm, zeros_hbm, out_hbm, ys_v, acc_v, sbuf, dbuf):
    c = lax.axis_index("c")
    s = lax.axis_index("s")
    wid = c * 16 + s
    pltpu.sync_copy(yst_hbm.at[pl.ds(wid * CPT, CPT)], ys_v)
    pltpu.sync_copy(zeros_hbm, acc_v)
    io = lax.broadcasted_iota(jnp.int32, (16,), 0)
    grot = [(cc + io) & 7 for cc in range(CPT)]
    arot = [(cc + io) & 15 for cc in range(CPT)]

    def do_group(g):
        s16 = sbuf[pl.ds(g * 16, 16)]
        d16 = dbuf[pl.ds(g * 16, 16)]
        for cc in range(CPT):
            v = plsc.load_gather(ys_v, [grot[cc], s16])
            plsc.addupdate_scatter(acc_v, [arot[cc], d16], v)

    def chunk(ch, carry):
        pltpu.sync_copy(src_hbm.at[pl.ds(ch * CH, CH)], sbuf)
        pltpu.sync_copy(dst_hbm.at[pl.ds(ch * CH, CH)], dbuf)

        def body(g, carry2):
            for u in range(GU):
                do_group(g * GU + u)
            return carry2

        lax.fori_loop(0, CH // (16 * GU), body, 0)
        return carry

    lax.fori_loop(0, E // CH, chunk, 0)

    def fold(j, carry):
        sl = pl.ds(j * 16, 16)
        for cc in range(CPT):
            acc_v[cc, sl] += acc_v[cc + CPT, sl]
        return carry

    lax.fori_loop(0, N // 16, fold, 0)
    pltpu.sync_copy(acc_v.at[pl.ds(0, CPT)], out_hbm.at[pl.ds(wid * CPT, CPT)])


def _dinv_diag(hist_blk):
    deg = jnp.sum(hist_blk, axis=0, keepdims=True) + 1.0
    dinv = lax.rsqrt(deg)
    r = lax.broadcasted_iota(jnp.int32, (BLK, BLK), 0)
    col = lax.broadcasted_iota(jnp.int32, (BLK, BLK), 1)
    return jnp.where(r == col, jnp.broadcast_to(dinv, (BLK, BLK)), 0.0)


_NT = (((1,), (1,)), ((), ()))


def _mm_t_body(xft_ref, a_ref, o_ref):
    o_ref[...] = lax.dot_general(xft_ref[...], a_ref[...], _NT,
                                 preferred_element_type=jnp.float32)


def _mm_t(xft, a):
    return pl.pallas_call(
        _mm_t_body,
        grid=(NBLK,),
        in_specs=[
            pl.BlockSpec((F, N), lambda i: (0, 0)),
            pl.BlockSpec((BLK, N), lambda i: (i, 0)),
        ],
        out_specs=pl.BlockSpec((F, BLK), lambda i: (0, i)),
        out_shape=jax.ShapeDtypeStruct((F, N), jnp.float32),
    )(xft, a)


def _stage_b_body(a1_ref, x1t_ref, xft_ref, hist_ref, w0t_ref, w1t_ref, o_ref):
    i = pl.program_id(0)
    x1t_full = x1t_ref[...]
    x1t_blk = x1t_ref[:, pl.ds(i * BLK, BLK)]
    x2t_blk = lax.dot_general(x1t_full, a1_ref[...], _NT,
                              preferred_element_type=jnp.float32)
    yt = (jnp.dot(w0t_ref[...], xft_ref[...] + x1t_blk,
                  preferred_element_type=jnp.float32)
          + jnp.dot(w1t_ref[...], x2t_blk, preferred_element_type=jnp.float32))
    o_ref[...] = jnp.dot(yt, _dinv_diag(hist_ref[...]),
                         preferred_element_type=jnp.float32)


def _stage_b(a1, x1t, xft, hist, w0t, w1t):
    return pl.pallas_call(
        _stage_b_body,
        grid=(NBLK,),
        in_specs=[
            pl.BlockSpec((BLK, N), lambda i: (i, 0)),
            pl.BlockSpec((F, N), lambda i: (0, 0)),
            pl.BlockSpec((F, BLK), lambda i: (0, i)),
            pl.BlockSpec((NW, BLK), lambda i: (0, i)),
            pl.BlockSpec((F, F), lambda i: (0, 0)),
            pl.BlockSpec((F, F), lambda i: (0, 0)),
        ],
        out_specs=pl.BlockSpec((F, BLK), lambda i: (0, i)),
        out_shape=jax.ShapeDtypeStruct((F, N), jnp.float32),
    )(a1, x1t, xft, hist, w0t, w1t)


def _combine_body(st_ref, yst_ref, hist_ref, b_ref, o_ref):
    sum_t = st_ref[...] + yst_ref[...]
    scaled = jnp.dot(sum_t, _dinv_diag(hist_ref[...]),
                     preferred_element_type=jnp.float32)
    o_ref[...] = jnp.transpose(scaled, (1, 0)) + b_ref[...]


def _combine(st, yst, hist, bias_row):
    return pl.pallas_call(
        _combine_body,
        grid=(NBLK,),
        in_specs=[
            pl.BlockSpec((F, BLK), lambda i: (0, i)),
            pl.BlockSpec((F, BLK), lambda i: (0, i)),
            pl.BlockSpec((NW, BLK), lambda i: (0, i)),
            pl.BlockSpec((1, F), lambda i: (0, 0)),
        ],
        out_specs=pl.BlockSpec((BLK, F), lambda i: (i, 0)),
        out_shape=jax.ShapeDtypeStruct((N, F), jnp.float32),
    )(st, yst, hist, bias_row)


def kernel(x, adj, cached_adj, Ws, bs):
    xft = x.reshape(N, F).T
    src = adj[0]
    dst = adj[1]
    eye_m = jnp.eye(M, dtype=jnp.float32)
    w0t = jnp.kron(eye_m, Ws[0].T)
    w1t = jnp.kron(eye_m, Ws[1].T)
    bias_row = jnp.tile(2.0 * bs[0] + bs[1], M).reshape(1, F)
    zeros_n = jnp.zeros((N,), jnp.float32)
    zeros_cpn = jnp.zeros((2 * CPT, N), jnp.float32)

    hist = _deg_k(dst, zeros_n)
    x1t = _mm_t(xft, cached_adj[0])
    yst = _stage_b(cached_adj[1], x1t, xft, hist, w0t, w1t)
    st = _agg_k(src, dst, yst, zeros_cpn)
    out = _combine(st, yst, hist, bias_row)
    return out.reshape(N, M, D)


# speedup vs baseline: 61.7368x; 1.3521x over previous
import functools

import jax
import jax.numpy as jnp
from jax import lax
from jax.experimental import pallas as pl
from jax.experimental.pallas import tpu as pltpu, tpu_sc as plsc

N = 4096
E = 65536
M = 2
D = 128
F = M * D
BLK = 256
NBLK = N // BLK
NW = 32
CPT = F // NW
EPW = E // NW
CH = 8192
GU = 4

_mesh = plsc.VectorSubcoreMesh(core_axis_name="c", subcore_axis_name="s")
_sc_params = pltpu.CompilerParams(needs_layout_passes=False)


@functools.partial(
    pl.kernel,
    out_type=jax.ShapeDtypeStruct((NW, N), jnp.float32),
    mesh=_mesh,
    compiler_params=_sc_params,
    scratch_types=[
        pltpu.VMEM((N,), jnp.float32),
        pltpu.VMEM((EPW,), jnp.int32),
    ],
)
def _deg_k(dst_hbm, zeros_hbm, out_hbm, hist_v, dbuf):
    c = lax.axis_index("c")
    s = lax.axis_index("s")
    wid = c * 16 + s
    pltpu.sync_copy(zeros_hbm, hist_v)
    pltpu.sync_copy(dst_hbm.at[pl.ds(wid * EPW, EPW)], dbuf)
    ones = jnp.ones((16,), jnp.float32)
    io = lax.broadcasted_iota(jnp.int32, (16,), 0)
    masks = [io == l for l in range(16)]

    def body(g, carry):
        d16 = dbuf[pl.ds(g * 16, 16)]
        for l in range(16):
            plsc.addupdate_scatter(hist_v, [d16], ones, mask=masks[l])
        return carry

    lax.fori_loop(0, EPW // 16, body, 0)
    pltpu.sync_copy(hist_v, out_hbm.at[wid])


@functools.partial(
    pl.kernel,
    out_type=jax.ShapeDtypeStruct((F, N), jnp.float32),
    mesh=_mesh,
    compiler_params=_sc_params,
    scratch_types=[
        pltpu.VMEM((CPT, N), jnp.float32),
        pltpu.VMEM((2 * CPT, N), jnp.float32),
        pltpu.VMEM((CH,), jnp.int32),
        pltpu.VMEM((CH,), jnp.int32),
    ],
)
def _agg_k(src_hbm, dst_hbm, yst_hbm, zeros_hbm, out_hbm, ys_v, acc_v, sbuf, dbuf):
    c = lax.axis_index("c")
    s = lax.axis_index("s")
    wid = c * 16 + s
    pltpu.sync_copy(yst_hbm.at[pl.ds(wid * CPT, CPT)], ys_v)
    pltpu.sync_copy(zeros_hbm, acc_v)
    io = lax.broadcasted_iota(jnp.int32, (16,), 0)
    grot = [(cc + io) & 7 for cc in range(CPT)]
    arot = [(cc + io) & 15 for cc in range(CPT)]

    def chunk(ch, carry):
        pltpu.sync_copy(src_hbm.at[pl.ds(ch * CH, CH)], sbuf)
        pltpu.sync_copy(dst_hbm.at[pl.ds(ch * CH, CH)], dbuf)

        def body(g, carry2):
            idx = [(sbuf[pl.ds((g * GU + u) * 16, 16)],
                    dbuf[pl.ds((g * GU + u) * 16, 16)]) for u in range(GU)]
            vals = [[plsc.load_gather(ys_v, [grot[cc], s16])
                     for cc in range(CPT)] for (s16, _) in idx]
            for u in range(GU):
                d16 = idx[u][1]
                for cc in range(CPT):
                    plsc.addupdate_scatter(acc_v, [arot[cc], d16], vals[u][cc])
            return carry2

        lax.fori_loop(0, CH // (16 * GU), body, 0)
        return carry

    lax.fori_loop(0, E // CH, chunk, 0)

    def fold(j, carry):
        sl = pl.ds(j * 16, 16)
        for cc in range(CPT):
            acc_v[cc, sl] += acc_v[cc + CPT, sl]
        return carry

    lax.fori_loop(0, N // 16, fold, 0)
    pltpu.sync_copy(acc_v.at[pl.ds(0, CPT)], out_hbm.at[pl.ds(wid * CPT, CPT)])


def _dinv_diag(hist_blk):
    deg = jnp.sum(hist_blk, axis=0, keepdims=True) + 1.0
    dinv = lax.rsqrt(deg)
    r = lax.broadcasted_iota(jnp.int32, (BLK, BLK), 0)
    col = lax.broadcasted_iota(jnp.int32, (BLK, BLK), 1)
    return jnp.where(r == col, jnp.broadcast_to(dinv, (BLK, BLK)), 0.0)


_NT = (((1,), (1,)), ((), ()))


def _mm_t_body(xft_ref, a_ref, o_ref):
    o_ref[...] = lax.dot_general(xft_ref[...], a_ref[...], _NT,
                                 preferred_element_type=jnp.float32)


def _mm_t(xft, a):
    return pl.pallas_call(
        _mm_t_body,
        grid=(NBLK,),
        in_specs=[
            pl.BlockSpec((F, N), lambda i: (0, 0)),
            pl.BlockSpec((BLK, N), lambda i: (i, 0)),
        ],
        out_specs=pl.BlockSpec((F, BLK), lambda i: (0, i)),
        out_shape=jax.ShapeDtypeStruct((F, N), jnp.float32),
    )(xft, a)


def _stage_b_body(a1_ref, x1t_ref, xft_ref, hist_ref, w0t_ref, w1t_ref, o_ref):
    i = pl.program_id(0)
    x1t_full = x1t_ref[...]
    x1t_blk = x1t_ref[:, pl.ds(i * BLK, BLK)]
    x2t_blk = lax.dot_general(x1t_full, a1_ref[...], _NT,
                              preferred_element_type=jnp.float32)
    yt = (jnp.dot(w0t_ref[...], xft_ref[...] + x1t_blk,
                  preferred_element_type=jnp.float32)
          + jnp.dot(w1t_ref[...], x2t_blk, preferred_element_type=jnp.float32))
    o_ref[...] = jnp.dot(yt, _dinv_diag(hist_ref[...]),
                         preferred_element_type=jnp.float32)


def _stage_b(a1, x1t, xft, hist, w0t, w1t):
    return pl.pallas_call(
        _stage_b_body,
        grid=(NBLK,),
        in_specs=[
            pl.BlockSpec((BLK, N), lambda i: (i, 0)),
            pl.BlockSpec((F, N), lambda i: (0, 0)),
            pl.BlockSpec((F, BLK), lambda i: (0, i)),
            pl.BlockSpec((NW, BLK), lambda i: (0, i)),
            pl.BlockSpec((F, F), lambda i: (0, 0)),
            pl.BlockSpec((F, F), lambda i: (0, 0)),
        ],
        out_specs=pl.BlockSpec((F, BLK), lambda i: (0, i)),
        out_shape=jax.ShapeDtypeStruct((F, N), jnp.float32),
    )(a1, x1t, xft, hist, w0t, w1t)


def _combine_body(st_ref, yst_ref, hist_ref, b_ref, o_ref):
    sum_t = st_ref[...] + yst_ref[...]
    scaled = jnp.dot(sum_t, _dinv_diag(hist_ref[...]),
                     preferred_element_type=jnp.float32)
    o_ref[...] = jnp.transpose(scaled, (1, 0)) + b_ref[...]


def _combine(st, yst, hist, bias_row):
    return pl.pallas_call(
        _combine_body,
        grid=(NBLK,),
        in_specs=[
            pl.BlockSpec((F, BLK), lambda i: (0, i)),
            pl.BlockSpec((F, BLK), lambda i: (0, i)),
            pl.BlockSpec((NW, BLK), lambda i: (0, i)),
            pl.BlockSpec((1, F), lambda i: (0, 0)),
        ],
        out_specs=pl.BlockSpec((BLK, F), lambda i: (i, 0)),
        out_shape=jax.ShapeDtypeStruct((N, F), jnp.float32),
    )(st, yst, hist, bias_row)


def kernel(x, adj, cached_adj, Ws, bs):
    xft = x.reshape(N, F).T
    src = adj[0]
    dst = adj[1]
    eye_m = jnp.eye(M, dtype=jnp.float32)
    w0t = jnp.kron(eye_m, Ws[0].T)
    w1t = jnp.kron(eye_m, Ws[1].T)
    bias_row = jnp.tile(2.0 * bs[0] + bs[1], M).reshape(1, F)
    zeros_n = jnp.zeros((N,), jnp.float32)
    zeros_cpn = jnp.zeros((2 * CPT, N), jnp.float32)

    hist = _deg_k(dst, zeros_n)
    x1t = _mm_t(xft, cached_adj[0])
    yst = _stage_b(cached_adj[1], x1t, xft, hist, w0t, w1t)
    st = _agg_k(src, dst, yst, zeros_cpn)
    out = _combine(st, yst, hist, bias_row)
    return out.reshape(N, M, D)
